# baseline (speedup 1.0000x reference)
import functools

import jax
import jax.numpy as jnp
from jax import lax
from jax.experimental import pallas as pl
from jax.experimental.pallas import tpu as pltpu
from jax.experimental.pallas import tpu_sc as plsc

_VOCAB = 1000000
_EMBED = 32
_BATCH = 16384
_HIST = 200
_HC = _HIST // 2

_NC = 2
_NS = 16
_NW = _NC * _NS

_ROWS_PER_W = _BATCH // _NW
_K = 4
_STEPS = _ROWS_PER_W // _K
_NB = 4
_ROUNDS = _STEPS // _NB
_TOKENS_H = _BATCH * _HC


@functools.partial(
    pl.kernel,
    out_type=jax.ShapeDtypeStruct((_TOKENS_H, _EMBED), jnp.float32),
    mesh=plsc.VectorSubcoreMesh(core_axis_name="c", subcore_axis_name="s"),
    scratch_types=[
        pltpu.VMEM((_NB, _K, _HC), jnp.int32),
        pltpu.VMEM((_NB, _K * _HC, _EMBED), jnp.float32),
        [pltpu.SemaphoreType.DMA] * _NB,
        [pltpu.SemaphoreType.DMA] * _NB,
    ],
    compiler_params=pltpu.CompilerParams(use_tc_tiling_on_sc=False),
)
def _gather_half(idx_hbm, table_hbm, out_hbm, idx_v, rows_v, gsem, wsem):
    wid = lax.axis_index("s") * _NC + lax.axis_index("c")
    base0 = wid * _ROWS_PER_W

    def fire_gathers(s, b):
        base = base0 + s * _K
        pltpu.sync_copy(idx_hbm.at[pl.ds(base, _K)], idx_v.at[b])
        for j in range(_K):
            pltpu.async_copy(
                table_hbm.at[idx_v.at[b, j]],
                rows_v.at[b, pl.ds(j * _HC, _HC)],
                gsem[b],
            )

    def drain_gathers(b):
        for j in range(_K):
            pltpu.make_async_copy(
                table_hbm.at[idx_v.at[b, j]],
                rows_v.at[b, pl.ds(j * _HC, _HC)],
                gsem[b],
            ).wait()

    def fire_write(s, b):
        base = base0 + s * _K
        pltpu.async_copy(
            rows_v.at[b], out_hbm.at[pl.ds(base * _HC, _K * _HC)], wsem[b]
        )

    def drain_write(b):
        pltpu.make_async_copy(
            rows_v.at[b], out_hbm.at[pl.ds(base0 * _HC, _K * _HC)], wsem[b]
        ).wait()

    def do_step(s, b, prefetch, prefetch_drains):
        if prefetch:
            b2 = (b + 2) % _NB
            if prefetch_drains:
                drain_write(b2)
            fire_gathers(s + 2, b2)
        drain_gathers(b)
        fire_write(s, b)

    fire_gathers(0, 0)
    fire_gathers(1, 1)

    for b in range(_NB):
        do_step(b, b, prefetch=True, prefetch_drains=(b >= 2))

    def round_body(t, carry):
        s0 = t * _NB
        for b in range(_NB):
            do_step(s0 + b, b, prefetch=True, prefetch_drains=True)
        return carry

    lax.fori_loop(1, _ROUNDS - 1, round_body, 0)

    s0 = (_ROUNDS - 1) * _NB
    for b in range(_NB):
        do_step(s0 + b, b, prefetch=(b < 2), prefetch_drains=True)

    for b in range(_NB):
        drain_write(b)


_BB = 128
_HW = _HC * _EMBED
_IN_ROWS = _BB * _HW // 128


def _transpose_body_first(x_ref, o_ref):
    o_ref[...] = x_ref[...].reshape(_BB, _HW).T


def _transpose_body_second(prev_ref, x_ref, o_ref):
    del prev_ref
    o_ref[...] = x_ref[...].reshape(_BB, _HW).T


_transpose_tc1 = pl.pallas_call(
    _transpose_body_first,
    grid=(_BATCH // _BB,),
    in_specs=[pl.BlockSpec((_IN_ROWS, 128), lambda i: (i, 0))],
    out_specs=pl.BlockSpec((_HW, _BB), lambda i: (0, i)),
    out_shape=jax.ShapeDtypeStruct((_HIST * _EMBED, _BATCH), jnp.float32),
)

_transpose_tc2 = pl.pallas_call(
    _transpose_body_second,
    grid=(_BATCH // _BB,),
    in_specs=[
        pl.BlockSpec(memory_space=pl.ANY),
        pl.BlockSpec((_IN_ROWS, 128), lambda i: (i, 0)),
    ],
    out_specs=pl.BlockSpec((_HW, _BB), lambda i: (1, i)),
    out_shape=jax.ShapeDtypeStruct((_HIST * _EMBED, _BATCH), jnp.float32),
    input_output_aliases={0: 0},
)


def kernel(input, table):
    idx = input.astype(jnp.int32)
    flat1 = _gather_half(idx[:, :_HC], table)
    flat2 = _gather_half(idx[:, _HC:], table)
    wide1 = flat1.reshape(_TOKENS_H * _EMBED // 128, 128)
    wide2 = flat2.reshape(_TOKENS_H * _EMBED // 128, 128)
    half = _transpose_tc1(wide1)
    out_t = _transpose_tc2(half, wide2)
    out_t = out_t.reshape(_HIST, _EMBED, _BATCH)
    return out_t.transpose(2, 0, 1)


# speedup vs baseline: 1.0351x; 1.0351x over previous
import functools

import jax
import jax.numpy as jnp
from jax import lax
from jax.experimental import pallas as pl
from jax.experimental.pallas import tpu as pltpu
from jax.experimental.pallas import tpu_sc as plsc

_VOCAB = 1000000
_EMBED = 32
_BATCH = 16384
_HIST = 200

_NC = 2
_NS = 16
_NW = _NC * _NS

_ROWS_PER_W = _BATCH // _NW
_K = 4
_STEPS = _ROWS_PER_W // _K
_NB = 4
_ROUNDS = _STEPS // _NB
_T = _HIST * _EMBED
_TOKENS = _BATCH * _HIST


@functools.partial(
    pl.kernel,
    out_type=jax.ShapeDtypeStruct((_TOKENS, _EMBED), jnp.float32),
    mesh=plsc.VectorSubcoreMesh(core_axis_name="c", subcore_axis_name="s"),
    scratch_types=[
        pltpu.VMEM((_NB, _K, _HIST), jnp.int32),
        pltpu.VMEM((_NB, _K * _HIST, _EMBED), jnp.float32),
        [pltpu.SemaphoreType.DMA] * _NB,
        [pltpu.SemaphoreType.DMA] * _NB,
    ],
    compiler_params=pltpu.CompilerParams(use_tc_tiling_on_sc=False),
)
def _gather_kernel(idx_hbm, table_hbm, out_hbm, idx_v, rows_v, gsem, wsem):
    wid = lax.axis_index("s") * _NC + lax.axis_index("c")
    base0 = wid * _ROWS_PER_W

    def fire_gathers(s, b):
        base = base0 + s * _K
        pltpu.sync_copy(idx_hbm.at[pl.ds(base, _K)], idx_v.at[b])
        for j in range(_K):
            pltpu.async_copy(
                table_hbm.at[idx_v.at[b, j]],
                rows_v.at[b, pl.ds(j * _HIST, _HIST)],
                gsem[b],
            )

    def drain_gathers(b):
        for j in range(_K):
            pltpu.make_async_copy(
                table_hbm.at[idx_v.at[b, j]],
                rows_v.at[b, pl.ds(j * _HIST, _HIST)],
                gsem[b],
            ).wait()

    def fire_write(s, b):
        base = base0 + s * _K
        pltpu.async_copy(
            rows_v.at[b], out_hbm.at[pl.ds(base * _HIST, _K * _HIST)], wsem[b]
        )

    def drain_write(b):
        pltpu.make_async_copy(
            rows_v.at[b], out_hbm.at[pl.ds(base0 * _HIST, _K * _HIST)], wsem[b]
        ).wait()

    def do_step(s, b, prefetch, prefetch_drains):
        if prefetch:
            b2 = (b + 2) % _NB
            if prefetch_drains:
                drain_write(b2)
            fire_gathers(s + 2, b2)
        drain_gathers(b)
        fire_write(s, b)

    fire_gathers(0, 0)
    fire_gathers(1, 1)

    for b in range(_NB):
        do_step(b, b, prefetch=True, prefetch_drains=(b >= 2))

    def round_body(t, carry):
        s0 = t * _NB
        for b in range(_NB):
            do_step(s0 + b, b, prefetch=True, prefetch_drains=True)
        return carry

    lax.fori_loop(1, _ROUNDS - 1, round_body, 0)

    s0 = (_ROUNDS - 1) * _NB
    for b in range(_NB):
        do_step(s0 + b, b, prefetch=(b < 2), prefetch_drains=True)

    for b in range(_NB):
        drain_write(b)


_BB = 128
_IN_ROWS = _BB * _T // 128


def _transpose_body(x_ref, o_ref):
    o_ref[...] = x_ref[...].reshape(_BB, _T).T


_transpose_tc = pl.pallas_call(
    _transpose_body,
    grid=(_BATCH // _BB,),
    in_specs=[pl.BlockSpec((_IN_ROWS, 128), lambda i: (i, 0))],
    out_specs=pl.BlockSpec((_T, _BB), lambda i: (0, i)),
    out_shape=jax.ShapeDtypeStruct((_T, _BATCH), jnp.float32),
)


def kernel(input, table):
    table_wide = lax.optimization_barrier(table.reshape(_VOCAB * _EMBED // 128, 128))
    flat = _gather_kernel(
        input.astype(jnp.int32), table_wide.reshape(_VOCAB, _EMBED)
    )
    wide = flat.reshape(_BATCH * _T // 128, 128)
    out_t = _transpose_tc(wide)
    out_t = out_t.reshape(_HIST, _EMBED, _BATCH)
    return out_t.transpose(2, 0, 1)
